# Initial kernel scaffold; baseline (speedup 1.0000x reference)
import jax
import jax.numpy as jnp
from jax.experimental import pallas as pl


def kernel(x, W_qkv, b_qkv, W_proj, b_proj, which, mask):
    raise NotImplementedError("write your pallas kernel here")


# speedup vs baseline: 1.9064x; 1.9064x over previous
import jax
import jax.numpy as jnp
from jax.experimental import pallas as pl
from jax.experimental.pallas import tpu as pltpu

_NVERT = 128
_D = 32
_DIM = 768
_H = 12
_HD = _DIM // _H
_W = 8
_ROWS = _NVERT * _D


def _qkv_body(x_ref, w_ref, b_ref, q_ref, k_ref, v_ref):
    acc = jnp.dot(x_ref[...], w_ref[...], preferred_element_type=jnp.float32)
    acc = acc + b_ref[...]
    q_ref[...] = acc[:, :_DIM] * jnp.float32(_HD ** 0.5)
    k_ref[...] = acc[:, _DIM:2 * _DIM]
    v_ref[...] = acc[:, 2 * _DIM:]


def _attn_body(which_ref, q_ref, k_ref, v_ref, m_ref, o_ref):
    n = pl.program_id(0)
    q = q_ref[...]
    ks, vs = [], []
    for w in range(_W):
        c = which_ref[n, w]
        ks.append(k_ref[pl.ds(c * _D, _D), :])
        vs.append(v_ref[pl.ds(c * _D, _D), :])
    kn = jnp.concatenate(ks, axis=0)
    vn = jnp.concatenate(vs, axis=0)
    mf = m_ref[0]
    neg = jnp.float32(-1e30)
    for h in range(_H):
        sl = slice(h * _HD, (h + 1) * _HD)
        qh = q[:, sl]
        kh = kn[:, sl]
        vh = vn[:, sl]
        s = jax.lax.dot_general(qh, kh, (((1,), (1,)), ((), ())),
                                preferred_element_type=jnp.float32)
        s = jnp.where(mf > 0, s, neg)
        mx = jnp.max(s, axis=-1, keepdims=True)
        p = jnp.exp(s - mx)
        denom = jnp.sum(p, axis=-1, keepdims=True)
        yh = jax.lax.dot_general(p, vh, (((1,), (0,)), ((), ())),
                                 preferred_element_type=jnp.float32)
        o_ref[:, sl] = yh / denom


def _proj_body(y_ref, w_ref, b_ref, o_ref):
    o_ref[...] = jnp.dot(y_ref[...], w_ref[...],
                         preferred_element_type=jnp.float32) + b_ref[...]


def kernel(x, W_qkv, b_qkv, W_proj, b_proj, which, mask):
    xm = x.reshape(_ROWS, _DIM)
    bm = 256
    grid_a = (_ROWS // bm,)
    q, k, v = pl.pallas_call(
        _qkv_body,
        grid=grid_a,
        in_specs=[
            pl.BlockSpec((bm, _DIM), lambda i: (i, 0)),
            pl.BlockSpec((_DIM, 3 * _DIM), lambda i: (0, 0)),
            pl.BlockSpec((1, 3 * _DIM), lambda i: (0, 0)),
        ],
        out_specs=[
            pl.BlockSpec((bm, _DIM), lambda i: (i, 0)),
            pl.BlockSpec((bm, _DIM), lambda i: (i, 0)),
            pl.BlockSpec((bm, _DIM), lambda i: (i, 0)),
        ],
        out_shape=[jax.ShapeDtypeStruct((_ROWS, _DIM), jnp.float32)] * 3,
    )(xm, W_qkv, b_qkv.reshape(1, 3 * _DIM))

    mf = mask.reshape(_NVERT, 1, _W * _D).astype(jnp.float32)
    y = pl.pallas_call(
        _attn_body,
        grid=(_NVERT,),
        in_specs=[
            pl.BlockSpec(memory_space=pltpu.SMEM),
            pl.BlockSpec((_D, _DIM), lambda n: (n, 0)),
            pl.BlockSpec((_ROWS, _DIM), lambda n: (0, 0)),
            pl.BlockSpec((_ROWS, _DIM), lambda n: (0, 0)),
            pl.BlockSpec((1, 1, _W * _D), lambda n: (n, 0, 0)),
        ],
        out_specs=pl.BlockSpec((_D, _DIM), lambda n: (n, 0)),
        out_shape=jax.ShapeDtypeStruct((_ROWS, _DIM), jnp.float32),
        compiler_params=pltpu.CompilerParams(
            vmem_limit_bytes=100 * 1024 * 1024,
        ),
    )(which, q, k, v, mf)

    out = pl.pallas_call(
        _proj_body,
        grid=(_ROWS // bm,),
        in_specs=[
            pl.BlockSpec((bm, _DIM), lambda i: (i, 0)),
            pl.BlockSpec((_DIM, _DIM), lambda i: (0, 0)),
            pl.BlockSpec((1, _DIM), lambda i: (0, 0)),
        ],
        out_specs=pl.BlockSpec((bm, _DIM), lambda i: (i, 0)),
        out_shape=jax.ShapeDtypeStruct((_ROWS, _DIM), jnp.float32),
    )(y, W_proj, b_proj.reshape(1, _DIM))

    return out.reshape(1, _NVERT, _D, _DIM)


# speedup vs baseline: 2.0119x; 1.0553x over previous
import jax
import jax.numpy as jnp
from jax.experimental import pallas as pl
from jax.experimental.pallas import tpu as pltpu

_NVERT = 128
_D = 32
_DIM = 768
_H = 12
_HD = _DIM // _H
_W = 8
_ROWS = _NVERT * _D


def _qkv_body(x_ref, w_ref, b_ref, q_ref, k_ref, v_ref):
    acc = jnp.dot(x_ref[...], w_ref[...], preferred_element_type=jnp.float32)
    acc = acc + b_ref[...]
    q_ref[...] = acc[:, :_DIM] * jnp.float32(_HD ** 0.5)
    k_ref[...] = acc[:, _DIM:2 * _DIM]
    v_ref[...] = acc[:, 2 * _DIM:]


_CB = 4


def _attn_body(which_ref, q_ref, k_ref, v_ref, m_ref, o_ref):
    n0 = pl.program_id(0) * _CB
    for c_i in range(_CB):
        n = n0 + c_i
        rs = slice(c_i * _D, (c_i + 1) * _D)
        q = q_ref[rs, :]
        ks, vs = [], []
        for w in range(_W):
            c = which_ref[n, w]
            ks.append(k_ref[pl.ds(c * _D, _D), :])
            vs.append(v_ref[pl.ds(c * _D, _D), :])
        kn = jnp.concatenate(ks, axis=0)
        vn = jnp.concatenate(vs, axis=0)
        madd = m_ref[c_i]
        for h in range(_H):
            sl = slice(h * _HD, (h + 1) * _HD)
            qh = q[:, sl]
            kh = kn[:, sl]
            vh = vn[:, sl]
            s = jax.lax.dot_general(qh, kh, (((1,), (1,)), ((), ())),
                                    preferred_element_type=jnp.float32)
            s = s + madd
            mx = jnp.max(s, axis=-1, keepdims=True)
            p = jnp.exp(s - mx)
            denom = jnp.sum(p, axis=-1, keepdims=True)
            yh = jax.lax.dot_general(p, vh, (((1,), (0,)), ((), ())),
                                     preferred_element_type=jnp.float32)
            o_ref[rs, sl] = yh / denom


def _proj_body(y_ref, w_ref, b_ref, o_ref):
    o_ref[...] = jnp.dot(y_ref[...], w_ref[...],
                         preferred_element_type=jnp.float32) + b_ref[...]


def kernel(x, W_qkv, b_qkv, W_proj, b_proj, which, mask):
    xm = x.reshape(_ROWS, _DIM)
    bm = 256
    grid_a = (_ROWS // bm,)
    q, k, v = pl.pallas_call(
        _qkv_body,
        grid=grid_a,
        in_specs=[
            pl.BlockSpec((bm, _DIM), lambda i: (i, 0)),
            pl.BlockSpec((_DIM, 3 * _DIM), lambda i: (0, 0)),
            pl.BlockSpec((1, 3 * _DIM), lambda i: (0, 0)),
        ],
        out_specs=[
            pl.BlockSpec((bm, _DIM), lambda i: (i, 0)),
            pl.BlockSpec((bm, _DIM), lambda i: (i, 0)),
            pl.BlockSpec((bm, _DIM), lambda i: (i, 0)),
        ],
        out_shape=[jax.ShapeDtypeStruct((_ROWS, _DIM), jnp.float32)] * 3,
    )(xm, W_qkv, b_qkv.reshape(1, 3 * _DIM))

    madd = jnp.where(mask, 0.0, -1e30).astype(jnp.float32)
    madd = madd.reshape(_NVERT, 1, _W * _D)
    y = pl.pallas_call(
        _attn_body,
        grid=(_NVERT // _CB,),
        in_specs=[
            pl.BlockSpec(memory_space=pltpu.SMEM),
            pl.BlockSpec((_CB * _D, _DIM), lambda n: (n, 0)),
            pl.BlockSpec((_ROWS, _DIM), lambda n: (0, 0)),
            pl.BlockSpec((_ROWS, _DIM), lambda n: (0, 0)),
            pl.BlockSpec((_CB, 1, _W * _D), lambda n: (n, 0, 0)),
        ],
        out_specs=pl.BlockSpec((_CB * _D, _DIM), lambda n: (n, 0)),
        out_shape=jax.ShapeDtypeStruct((_ROWS, _DIM), jnp.float32),
        compiler_params=pltpu.CompilerParams(
            vmem_limit_bytes=100 * 1024 * 1024,
        ),
    )(which, q, k, v, madd)

    out = pl.pallas_call(
        _proj_body,
        grid=(_ROWS // bm,),
        in_specs=[
            pl.BlockSpec((bm, _DIM), lambda i: (i, 0)),
            pl.BlockSpec((_DIM, _DIM), lambda i: (0, 0)),
            pl.BlockSpec((1, _DIM), lambda i: (0, 0)),
        ],
        out_specs=pl.BlockSpec((bm, _DIM), lambda i: (i, 0)),
        out_shape=jax.ShapeDtypeStruct((_ROWS, _DIM), jnp.float32),
    )(y, W_proj, b_proj.reshape(1, _DIM))

    return out.reshape(1, _NVERT, _D, _DIM)


# speedup vs baseline: 5.5536x; 2.7604x over previous
import jax
import jax.numpy as jnp
from jax.experimental import pallas as pl
from jax.experimental.pallas import tpu as pltpu

_NVERT = 128
_D = 32
_DIM = 768
_H = 12
_HD = _DIM // _H
_W = 8
_WD = _W * _D
_ROWS = _NVERT * _D
_BM = 256
_CB = _BM // _D
_NT = _ROWS // _BM


def _body(which_ref, x_ref, wqkv_ref, bqkv_ref, m_ref, wproj_ref, bproj_ref,
          o_ref, q_s, k_s, v_s, kn_s, vn_s, s_s, p_s, y_s):
    i = pl.program_id(0)

    @pl.when(i < _NT)
    def _qkv():
        rows = pl.ds(i * _BM, _BM)
        acc = jnp.dot(x_ref[...], wqkv_ref[...],
                      preferred_element_type=jnp.float32) + bqkv_ref[...]
        q_s[rows, :] = acc[:, :_DIM] * jnp.float32(_HD ** 0.5)
        k_s[rows, :] = acc[:, _DIM:2 * _DIM]
        v_s[rows, :] = acc[:, 2 * _DIM:].astype(jnp.bfloat16)

    @pl.when(i >= _NT)
    def _attn():
        j = i - _NT
        n0 = j * _CB
        qrows = pl.ds(j * _BM, _BM)
        for c_i in range(_CB):
            n = n0 + c_i
            for w in range(_W):
                c = which_ref[n, w]
                dst = pl.ds((c_i * _W + w) * _D, _D)
                src = pl.ds(c * _D, _D)
                kn_s[dst, :] = k_s[src, :]
                vn_s[dst, :] = v_s[src, :]
        q = q_s[qrows, :]
        for c_i in range(_CB):
            rs = slice(c_i * _D, (c_i + 1) * _D)
            krs = slice(c_i * _WD, (c_i + 1) * _WD)
            madd = m_ref[c_i]
            for h in range(_H):
                sl = slice(h * _HD, (h + 1) * _HD)
                s = jax.lax.dot_general(q[rs, sl], kn_s[krs, sl],
                                        (((1,), (1,)), ((), ())),
                                        preferred_element_type=jnp.float32)
                s_s[rs, h * _WD:(h + 1) * _WD] = s + madd
        for h in range(_H):
            cs = slice(h * _WD, (h + 1) * _WD)
            s = s_s[:, cs]
            mx = jnp.max(s, axis=-1, keepdims=True)
            p = jnp.exp(s - mx)
            denom = jnp.sum(p, axis=-1, keepdims=True)
            p_s[:, cs] = (p * (1.0 / denom)).astype(jnp.bfloat16)
        for c_i in range(_CB):
            rs = slice(c_i * _D, (c_i + 1) * _D)
            krs = slice(c_i * _WD, (c_i + 1) * _WD)
            for h in range(_H):
                sl = slice(h * _HD, (h + 1) * _HD)
                p = p_s[rs, h * _WD:(h + 1) * _WD]
                y_s[rs, sl] = jax.lax.dot_general(
                    p, vn_s[krs, sl], (((1,), (0,)), ((), ())),
                    preferred_element_type=jnp.float32).astype(jnp.bfloat16)
        o_ref[...] = jnp.dot(y_s[...], wproj_ref[...],
                             preferred_element_type=jnp.float32) + bproj_ref[...]


def kernel(x, W_qkv, b_qkv, W_proj, b_proj, which, mask):
    xm = x.reshape(_ROWS, _DIM)
    madd = jnp.where(mask, 0.0, -1e30).astype(jnp.float32)
    madd = madd.reshape(_NVERT, 1, _WD)
    wproj_bf = W_proj.astype(jnp.bfloat16)

    out = pl.pallas_call(
        _body,
        grid=(2 * _NT,),
        in_specs=[
            pl.BlockSpec(memory_space=pltpu.SMEM),
            pl.BlockSpec((_BM, _DIM), lambda i: (jnp.minimum(i, _NT - 1), 0)),
            pl.BlockSpec((_DIM, 3 * _DIM), lambda i: (0, 0)),
            pl.BlockSpec((1, 3 * _DIM), lambda i: (0, 0)),
            pl.BlockSpec((_CB, 1, _WD),
                         lambda i: (jnp.maximum(i - _NT, 0), 0, 0)),
            pl.BlockSpec((_DIM, _DIM), lambda i: (0, 0)),
            pl.BlockSpec((1, _DIM), lambda i: (0, 0)),
        ],
        out_specs=pl.BlockSpec((_BM, _DIM), lambda i: (jnp.maximum(i - _NT, 0), 0)),
        out_shape=jax.ShapeDtypeStruct((_ROWS, _DIM), jnp.float32),
        scratch_shapes=[
            pltpu.VMEM((_ROWS, _DIM), jnp.float32),
            pltpu.VMEM((_ROWS, _DIM), jnp.float32),
            pltpu.VMEM((_ROWS, _DIM), jnp.bfloat16),
            pltpu.VMEM((_CB * _WD, _DIM), jnp.float32),
            pltpu.VMEM((_CB * _WD, _DIM), jnp.bfloat16),
            pltpu.VMEM((_BM, _H * _WD), jnp.float32),
            pltpu.VMEM((_BM, _H * _WD), jnp.bfloat16),
            pltpu.VMEM((_BM, _DIM), jnp.bfloat16),
        ],
        compiler_params=pltpu.CompilerParams(
            vmem_limit_bytes=110 * 1024 * 1024,
        ),
    )(which, xm, W_qkv, b_qkv.reshape(1, 3 * _DIM), madd,
      wproj_bf, b_proj.reshape(1, _DIM))

    return out.reshape(1, _NVERT, _D, _DIM)


# speedup vs baseline: 5.6550x; 1.0183x over previous
import jax
import jax.numpy as jnp
from jax.experimental import pallas as pl
from jax.experimental.pallas import tpu as pltpu

_NVERT = 128
_D = 32
_DIM = 768
_H = 12
_HD = _DIM // _H
_W = 8
_WD = _W * _D
_ROWS = _NVERT * _D
_BM = 256
_CB = _BM // _D
_NT = _ROWS // _BM


def _body(which_ref, x_ref, wqk_ref, wv_ref, bqkv_ref, m_ref, wproj_ref,
          bproj_ref, o_ref, q_s, k_s, v_s, kn_s, vn_s, s_s, p_s, y_s):
    i = pl.program_id(0)

    @pl.when(i < _NT)
    def _qkv():
        rows = pl.ds(i * _BM, _BM)
        x = x_ref[...]
        acc = jnp.dot(x, wqk_ref[...],
                      preferred_element_type=jnp.float32) + bqkv_ref[:, :2 * _DIM]
        accv = jnp.dot(x.astype(jnp.bfloat16), wv_ref[...],
                       preferred_element_type=jnp.float32) + bqkv_ref[:, 2 * _DIM:]
        q_s[rows, :] = acc[:, :_DIM] * jnp.float32(_HD ** 0.5)
        k_s[rows, :] = acc[:, _DIM:2 * _DIM]
        v_s[rows, :] = accv.astype(jnp.bfloat16)

    @pl.when(i >= _NT)
    def _attn():
        j = i - _NT
        n0 = j * _CB
        qrows = pl.ds(j * _BM, _BM)
        for c_i in range(_CB):
            n = n0 + c_i
            for w in range(_W):
                c = which_ref[n, w]
                dst = pl.ds((c_i * _W + w) * _D, _D)
                src = pl.ds(c * _D, _D)
                kn_s[dst, :] = k_s[src, :]
                vn_s[dst, :] = v_s[src, :]
        q = q_s[qrows, :]
        for c_i in range(_CB):
            rs = slice(c_i * _D, (c_i + 1) * _D)
            krs = slice(c_i * _WD, (c_i + 1) * _WD)
            madd = m_ref[c_i]
            for h in range(_H):
                sl = slice(h * _HD, (h + 1) * _HD)
                s = jax.lax.dot_general(q[rs, sl], kn_s[krs, sl],
                                        (((1,), (1,)), ((), ())),
                                        preferred_element_type=jnp.float32)
                s_s[rs, h * _WD:(h + 1) * _WD] = s + madd
        recips = []
        for h in range(_H):
            cs = slice(h * _WD, (h + 1) * _WD)
            s = s_s[:, cs]
            mx = jnp.max(s, axis=-1, keepdims=True)
            p = jnp.exp(s - mx)
            denom = jnp.sum(p, axis=-1, keepdims=True)
            recips.append(1.0 / denom)
            p_s[:, cs] = p.astype(jnp.bfloat16)
        for c_i in range(_CB):
            rs = slice(c_i * _D, (c_i + 1) * _D)
            krs = slice(c_i * _WD, (c_i + 1) * _WD)
            for h in range(_H):
                sl = slice(h * _HD, (h + 1) * _HD)
                p = p_s[rs, h * _WD:(h + 1) * _WD]
                yh = jax.lax.dot_general(
                    p, vn_s[krs, sl], (((1,), (0,)), ((), ())),
                    preferred_element_type=jnp.float32)
                y_s[rs, sl] = (yh * recips[h][rs]).astype(jnp.bfloat16)
        o_ref[...] = jnp.dot(y_s[...], wproj_ref[...],
                             preferred_element_type=jnp.float32) + bproj_ref[...]


def kernel(x, W_qkv, b_qkv, W_proj, b_proj, which, mask):
    xm = x.reshape(_ROWS, _DIM)
    madd = jnp.where(mask, 0.0, -1e30).astype(jnp.float32)
    madd = madd.reshape(_NVERT, 1, _WD)
    wproj_bf = W_proj.astype(jnp.bfloat16)

    out = pl.pallas_call(
        _body,
        grid=(2 * _NT,),
        in_specs=[
            pl.BlockSpec(memory_space=pltpu.SMEM),
            pl.BlockSpec((_BM, _DIM), lambda i: (jnp.minimum(i, _NT - 1), 0)),
            pl.BlockSpec((_DIM, 2 * _DIM), lambda i: (0, 0)),
            pl.BlockSpec((_DIM, _DIM), lambda i: (0, 0)),
            pl.BlockSpec((1, 3 * _DIM), lambda i: (0, 0)),
            pl.BlockSpec((_CB, 1, _WD),
                         lambda i: (jnp.maximum(i - _NT, 0), 0, 0)),
            pl.BlockSpec((_DIM, _DIM), lambda i: (0, 0)),
            pl.BlockSpec((1, _DIM), lambda i: (0, 0)),
        ],
        out_specs=pl.BlockSpec((_BM, _DIM), lambda i: (jnp.maximum(i - _NT, 0), 0)),
        out_shape=jax.ShapeDtypeStruct((_ROWS, _DIM), jnp.float32),
        scratch_shapes=[
            pltpu.VMEM((_ROWS, _DIM), jnp.float32),
            pltpu.VMEM((_ROWS, _DIM), jnp.float32),
            pltpu.VMEM((_ROWS, _DIM), jnp.bfloat16),
            pltpu.VMEM((_CB * _WD, _DIM), jnp.float32),
            pltpu.VMEM((_CB * _WD, _DIM), jnp.bfloat16),
            pltpu.VMEM((_BM, _H * _WD), jnp.float32),
            pltpu.VMEM((_BM, _H * _WD), jnp.bfloat16),
            pltpu.VMEM((_BM, _DIM), jnp.bfloat16),
        ],
        compiler_params=pltpu.CompilerParams(
            vmem_limit_bytes=110 * 1024 * 1024,
        ),
    )(which, xm, W_qkv[:, :2 * _DIM], W_qkv[:, 2 * _DIM:].astype(jnp.bfloat16),
      b_qkv.reshape(1, 3 * _DIM), madd, wproj_bf, b_proj.reshape(1, _DIM))

    return out.reshape(1, _NVERT, _D, _DIM)
